# baseline (speedup 1.0000x reference)
import functools

import jax
import jax.numpy as jnp
from jax import lax
from jax.experimental import pallas as pl
from jax.experimental.pallas import tpu as pltpu
from jax.experimental.pallas import tpu_sc as plsc

_IGNORE = 255
_B, _C, _H, _W = 8, 19, 512, 512
_HB = 256
_N = _B * _H * _W
_K = int(0.2 * _N)
_GB, _GH = _B, _H // _HB
_SROWS, _SCOLS = _H, _B * _W
_NWORK = 32
_CHUNK = _N // _NWORK
_NBINS = 256


def _ce_kernel(logits_ref, labels_ref, val_ref, p16_ref):
    lab = labels_ref[0]
    m = logits_ref[0, 0]
    for c in range(1, _C):
        m = jnp.maximum(m, logits_ref[0, c])
    s = jnp.zeros((_HB, _W), jnp.float32)
    picked = jnp.zeros((_HB, _W), jnp.float32)
    for c in range(_C):
        xc = logits_ref[0, c]
        s = s + jnp.exp(xc - m)
        picked = jnp.where(lab == c, xc, picked)
    nll = (m - picked) + jnp.log(s)
    loss = jnp.where(lab != _IGNORE, nll, 0.0)
    val_ref[:, :] = loss
    p16_ref[:, :] = pltpu.bitcast(loss.astype(jnp.bfloat16), jnp.int16)


def _sc_hist(p16_hbm, out_hbm, buf, hist):
    w = lax.axis_index("s") * 2 + lax.axis_index("c")
    iota = lax.broadcasted_iota(jnp.int32, (16,), 0)
    zeros16 = jnp.zeros((16,), jnp.int32)
    ones16 = jnp.ones((16,), jnp.int32)
    for j in range(_NBINS):
        hist[pl.ds(j * 16, 16)] = zeros16
    pltpu.sync_copy(p16_hbm.at[pl.ds(w * (_CHUNK // 2), _CHUNK // 2)], buf)

    def body(i, carry):
        v32 = buf[pl.ds(i * 16, 16)]
        lo = v32 & 0xFFFF
        hi = lax.shift_right_logical(v32, 16)
        plsc.addupdate_scatter(
            hist, [iota * _NBINS + lax.shift_right_logical(lo, 7)], ones16)
        plsc.addupdate_scatter(
            hist, [iota * _NBINS + lax.shift_right_logical(hi, 7)], ones16)
        return carry

    lax.fori_loop(0, _CHUNK // 32, body, 0)
    pltpu.sync_copy(hist, out_hbm.at[w])


def _resolve_kernel(hists_ref, p16_ref, val_ref, out_ref):
    kf = jnp.float32(_K)
    one16 = jnp.int16(1)

    h = jnp.sum(hists_ref[:, :].astype(jnp.float32), axis=0, keepdims=True)
    bins = jnp.zeros((1, _NBINS), jnp.float32)
    for l in range(16):
        bins = bins + h[:, l * _NBINS:(l + 1) * _NBINS]
    jvec = lax.broadcasted_iota(jnp.int32, (1, _NBINS), 1)

    def cnt_above(j):
        return jnp.sum(jnp.where(jvec > j, bins, 0.0))

    def bucket_body(_, carry):
        blo, bhi = carry
        mid = blo + (bhi - blo) // 2
        below = cnt_above(mid) < kf
        active = blo < bhi
        new_hi = jnp.where(active & below, mid, bhi)
        new_lo = jnp.where(active & (~below), mid + 1, blo)
        return new_lo, new_hi

    bb, _ = lax.fori_loop(0, 8, bucket_body,
                          (jnp.int32(0), jnp.int32(_NBINS - 1)))

    def count_gt(mid):
        mid16 = mid.astype(jnp.int16)
        acc = jnp.zeros((16, _SCOLS), jnp.int16)
        for j in range(_SROWS // 16):
            blk = p16_ref[pl.ds(j * 16, 16), :]
            acc = acc + jnp.where(blk > mid16, one16, jnp.int16(0))
        return jnp.sum(acc.astype(jnp.float32))

    def body(_, carry):
        lo, hi = carry
        mid = lo + (hi - lo) // 2
        cnt = count_gt(mid)
        active = lo < hi
        below = cnt < kf
        new_hi = jnp.where(active & below, mid, hi)
        new_lo = jnp.where(active & (~below), mid + 1, lo)
        return new_lo, new_hi

    lo, _hi = lax.fori_loop(0, 7, body, (bb * 128, bb * 128 + 127))
    t16 = lo.astype(jnp.int16)

    zf = jnp.zeros((8, _SCOLS), jnp.float32)
    zi = jnp.zeros((8, _SCOLS), jnp.int16)
    s_gt, s_eq = zf, zf
    c_gt16, c_eq16 = zi, zi
    for j in range(_SROWS // 8):
        blk = p16_ref[pl.ds(j * 8, 8), :]
        v = val_ref[pl.ds(j * 8, 8), :]
        gt_i = jnp.where(blk > t16, one16, jnp.int16(0))
        eq_i = jnp.where(blk == t16, one16, jnp.int16(0))
        c_gt16 = c_gt16 + gt_i
        c_eq16 = c_eq16 + eq_i
        s_gt = s_gt + gt_i.astype(jnp.float32) * v
        s_eq = s_eq + eq_i.astype(jnp.float32) * v
    sum_gt = jnp.sum(s_gt)
    cnt_gt = jnp.sum(c_gt16.astype(jnp.float32))
    sum_eq = jnp.sum(s_eq)
    cnt_eq = jnp.maximum(jnp.sum(c_eq16.astype(jnp.float32)), 1.0)
    kth = sum_eq / cnt_eq
    out_ref[0, 0] = (sum_gt + (kf - cnt_gt) * kth) / kf


def kernel(logits, labels):
    val, p16 = pl.pallas_call(
        _ce_kernel,
        grid=(_GB, _GH),
        in_specs=[
            pl.BlockSpec((1, _C, _HB, _W), lambda b, h: (b, 0, h, 0)),
            pl.BlockSpec((1, _HB, _W), lambda b, h: (b, h, 0)),
        ],
        out_specs=[
            pl.BlockSpec((_HB, _W), lambda b, h: (h, b)),
            pl.BlockSpec((_HB, _W), lambda b, h: (h, b)),
        ],
        out_shape=[
            jax.ShapeDtypeStruct((_SROWS, _SCOLS), jnp.float32),
            jax.ShapeDtypeStruct((_SROWS, _SCOLS), jnp.int16),
        ],
    )(logits, labels)

    mesh = plsc.VectorSubcoreMesh(core_axis_name="c", subcore_axis_name="s")
    sc_hist = functools.partial(
        pl.kernel,
        mesh=mesh,
        compiler_params=pltpu.CompilerParams(needs_layout_passes=False),
        out_type=jax.ShapeDtypeStruct((_NWORK, _NBINS * 16), jnp.int32),
        scratch_types=[
            pltpu.VMEM((_CHUNK // 2,), jnp.int32),
            pltpu.VMEM((_NBINS * 16,), jnp.int32),
        ],
    )(_sc_hist)
    p32 = lax.bitcast_convert_type(
        p16.reshape(-1, 2), jnp.int32)
    hists = sc_hist(p32)

    out = pl.pallas_call(
        _resolve_kernel,
        out_specs=pl.BlockSpec(memory_space=pltpu.SMEM),
        out_shape=jax.ShapeDtypeStruct((1, 1), jnp.float32),
    )(hists, p16, val)
    return out[0, 0]


# speedup vs baseline: 1.0003x; 1.0003x over previous
import functools

import jax
import jax.numpy as jnp
from jax import lax
from jax.experimental import pallas as pl
from jax.experimental.pallas import tpu as pltpu
from jax.experimental.pallas import tpu_sc as plsc

_IGNORE = 255
_B, _C, _H, _W = 8, 19, 512, 512
_HB = 256
_N = _B * _H * _W
_K = int(0.2 * _N)
_GB, _GH = _B, _H // _HB
_SROWS, _SCOLS = _H, _B * _W
_NWORK = 32
_CHUNK = _N // _NWORK
_NBINS = 256


def _ce_kernel(logits_ref, labels_ref, val_ref, p16_ref):
    lab = labels_ref[0]
    m = logits_ref[0, 0]
    for c in range(1, _C):
        m = jnp.maximum(m, logits_ref[0, c])
    s = jnp.zeros((_HB, _W), jnp.float32)
    picked = jnp.zeros((_HB, _W), jnp.float32)
    for c in range(_C):
        xc = logits_ref[0, c]
        s = s + jnp.exp(xc - m)
        picked = jnp.where(lab == c, xc, picked)
    nll = (m - picked) + jnp.log(s)
    loss = jnp.where(lab != _IGNORE, nll, 0.0)
    val_ref[:, :] = loss
    p16_ref[:, :] = pltpu.bitcast(loss.astype(jnp.bfloat16), jnp.int16)


def _sc_hist(p16_hbm, out_hbm, buf, hist):
    w = lax.axis_index("s") * 2 + lax.axis_index("c")
    iota = lax.broadcasted_iota(jnp.int32, (16,), 0)
    zeros16 = jnp.zeros((16,), jnp.int32)
    ones16 = jnp.ones((16,), jnp.int32)
    for j in range(_NBINS):
        hist[pl.ds(j * 16, 16)] = zeros16
    pltpu.sync_copy(p16_hbm.at[pl.ds(w * (_CHUNK // 2), _CHUNK // 2)], buf)

    def body(i, carry):
        for u in range(8):
            v32 = buf[pl.ds(i * 128 + u * 16, 16)]
            lo = v32 & 0xFFFF
            hi = lax.shift_right_logical(v32, 16)
            plsc.addupdate_scatter(
                hist, [iota * _NBINS + lax.shift_right_logical(lo, 7)], ones16)
            plsc.addupdate_scatter(
                hist, [iota * _NBINS + lax.shift_right_logical(hi, 7)], ones16)
        return carry

    lax.fori_loop(0, _CHUNK // 256, body, 0)
    pltpu.sync_copy(hist, out_hbm.at[w])


def _resolve_kernel(hists_ref, p16_ref, val_ref, out_ref):
    kf = jnp.float32(_K)
    one16 = jnp.int16(1)

    h = jnp.sum(hists_ref[:, :].astype(jnp.float32), axis=0, keepdims=True)
    bins = jnp.zeros((1, _NBINS), jnp.float32)
    for l in range(16):
        bins = bins + h[:, l * _NBINS:(l + 1) * _NBINS]
    jvec = lax.broadcasted_iota(jnp.int32, (1, _NBINS), 1)

    def cnt_above(j):
        return jnp.sum(jnp.where(jvec > j, bins, 0.0))

    def bucket_body(_, carry):
        blo, bhi = carry
        mid = blo + (bhi - blo) // 2
        below = cnt_above(mid) < kf
        active = blo < bhi
        new_hi = jnp.where(active & below, mid, bhi)
        new_lo = jnp.where(active & (~below), mid + 1, blo)
        return new_lo, new_hi

    bb, _ = lax.fori_loop(0, 8, bucket_body,
                          (jnp.int32(0), jnp.int32(_NBINS - 1)))

    def count_gt(mid):
        mid16 = mid.astype(jnp.int16)
        acc = jnp.zeros((16, _SCOLS), jnp.int16)
        for j in range(_SROWS // 16):
            blk = p16_ref[pl.ds(j * 16, 16), :]
            acc = acc + jnp.where(blk > mid16, one16, jnp.int16(0))
        return jnp.sum(acc.astype(jnp.float32))

    def body(_, carry):
        lo, hi = carry
        mid = lo + (hi - lo) // 2
        cnt = count_gt(mid)
        active = lo < hi
        below = cnt < kf
        new_hi = jnp.where(active & below, mid, hi)
        new_lo = jnp.where(active & (~below), mid + 1, lo)
        return new_lo, new_hi

    lo, _hi = lax.fori_loop(0, 7, body, (bb * 128, bb * 128 + 127))
    t16 = lo.astype(jnp.int16)

    zf = jnp.zeros((8, _SCOLS), jnp.float32)
    zi = jnp.zeros((8, _SCOLS), jnp.int16)
    s_gt, s_eq = zf, zf
    c_gt16, c_eq16 = zi, zi
    for j in range(_SROWS // 8):
        blk = p16_ref[pl.ds(j * 8, 8), :]
        v = val_ref[pl.ds(j * 8, 8), :]
        gt_i = jnp.where(blk > t16, one16, jnp.int16(0))
        eq_i = jnp.where(blk == t16, one16, jnp.int16(0))
        c_gt16 = c_gt16 + gt_i
        c_eq16 = c_eq16 + eq_i
        s_gt = s_gt + gt_i.astype(jnp.float32) * v
        s_eq = s_eq + eq_i.astype(jnp.float32) * v
    sum_gt = jnp.sum(s_gt)
    cnt_gt = jnp.sum(c_gt16.astype(jnp.float32))
    sum_eq = jnp.sum(s_eq)
    cnt_eq = jnp.maximum(jnp.sum(c_eq16.astype(jnp.float32)), 1.0)
    kth = sum_eq / cnt_eq
    out_ref[0, 0] = (sum_gt + (kf - cnt_gt) * kth) / kf


def kernel(logits, labels):
    val, p16 = pl.pallas_call(
        _ce_kernel,
        grid=(_GB, _GH),
        in_specs=[
            pl.BlockSpec((1, _C, _HB, _W), lambda b, h: (b, 0, h, 0)),
            pl.BlockSpec((1, _HB, _W), lambda b, h: (b, h, 0)),
        ],
        out_specs=[
            pl.BlockSpec((_HB, _W), lambda b, h: (h, b)),
            pl.BlockSpec((_HB, _W), lambda b, h: (h, b)),
        ],
        out_shape=[
            jax.ShapeDtypeStruct((_SROWS, _SCOLS), jnp.float32),
            jax.ShapeDtypeStruct((_SROWS, _SCOLS), jnp.int16),
        ],
    )(logits, labels)

    mesh = plsc.VectorSubcoreMesh(core_axis_name="c", subcore_axis_name="s")
    sc_hist = functools.partial(
        pl.kernel,
        mesh=mesh,
        compiler_params=pltpu.CompilerParams(needs_layout_passes=False),
        out_type=jax.ShapeDtypeStruct((_NWORK, _NBINS * 16), jnp.int32),
        scratch_types=[
            pltpu.VMEM((_CHUNK // 2,), jnp.int32),
            pltpu.VMEM((_NBINS * 16,), jnp.int32),
        ],
    )(_sc_hist)
    p32 = lax.bitcast_convert_type(
        p16.reshape(-1, 2), jnp.int32)
    hists = sc_hist(p32)

    out = pl.pallas_call(
        _resolve_kernel,
        out_specs=pl.BlockSpec(memory_space=pltpu.SMEM),
        out_shape=jax.ShapeDtypeStruct((1, 1), jnp.float32),
    )(hists, p16, val)
    return out[0, 0]


# speedup vs baseline: 16.3077x; 16.3025x over previous
import jax
import jax.numpy as jnp
from jax.experimental import pallas as pl
from jax.experimental.pallas import tpu as pltpu

_IGNORE = 255
_B, _C, _H, _W = 8, 19, 512, 512
_HB = 256
_N = _B * _H * _W
_K = int(0.2 * _N)
_GB, _GH = _B, _H // _HB
_SROWS, _SCOLS = _H, _B * _W


def _ce_topk_kernel(logits_ref, labels_ref, out_ref, val_ref, p16_ref):
    b = pl.program_id(0)
    h = pl.program_id(1)

    lab = labels_ref[0]

    m = logits_ref[0, 0]
    for c in range(1, _C):
        m = jnp.maximum(m, logits_ref[0, c])
    s = jnp.zeros((_HB, _W), jnp.float32)
    picked = jnp.zeros((_HB, _W), jnp.float32)
    for c in range(_C):
        xc = logits_ref[0, c]
        s = s + jnp.exp(xc - m)
        picked = jnp.where(lab == c, xc, picked)
    nll = (m - picked) + jnp.log(s)
    loss = jnp.where(lab != _IGNORE, nll, 0.0)

    val_ref[pl.ds(h * _HB, _HB), pl.ds(b * _W, _W)] = loss
    p16_ref[pl.ds(h * _HB, _HB), pl.ds(b * _W, _W)] = pltpu.bitcast(
        loss.astype(jnp.bfloat16), jnp.int16)

    @pl.when((b == _GB - 1) & (h == _GH - 1))
    def _select():
        kf = jnp.float32(_K)
        one16 = jnp.int16(1)

        def count_gt(mid):
            mid16 = mid.astype(jnp.int16)
            acc = jnp.zeros((16, _SCOLS), jnp.int16)
            for j in range(_SROWS // 16):
                blk = p16_ref[pl.ds(j * 16, 16), :]
                acc = acc + jnp.where(blk > mid16, one16, jnp.int16(0))
            return jnp.sum(acc.astype(jnp.float32))

        def body(_, carry):
            lo, hi = carry
            mid = lo + (hi - lo) // 2
            cnt = count_gt(mid)
            active = lo < hi
            below = cnt < kf
            new_hi = jnp.where(active & below, mid, hi)
            new_lo = jnp.where(active & (~below), mid + 1, lo)
            return new_lo, new_hi

        lo, _hi = jax.lax.fori_loop(
            0, 15, body, (jnp.int32(0), jnp.int32(0x7F80)))
        t16 = lo.astype(jnp.int16)

        zf = jnp.zeros((8, _SCOLS), jnp.float32)
        zi = jnp.zeros((8, _SCOLS), jnp.int16)
        s_gt, s_eq = zf, zf
        c_gt16, c_eq16 = zi, zi
        for j in range(_SROWS // 8):
            blk = p16_ref[pl.ds(j * 8, 8), :]
            v = val_ref[pl.ds(j * 8, 8), :]
            gt_i = jnp.where(blk > t16, one16, jnp.int16(0))
            eq_i = jnp.where(blk == t16, one16, jnp.int16(0))
            c_gt16 = c_gt16 + gt_i
            c_eq16 = c_eq16 + eq_i
            s_gt = s_gt + gt_i.astype(jnp.float32) * v
            s_eq = s_eq + eq_i.astype(jnp.float32) * v
        sum_gt = jnp.sum(s_gt)
        cnt_gt = jnp.sum(c_gt16.astype(jnp.float32))
        sum_eq = jnp.sum(s_eq)
        cnt_eq = jnp.maximum(jnp.sum(c_eq16.astype(jnp.float32)), 1.0)
        kth = sum_eq / cnt_eq
        out_ref[0, 0] = (sum_gt + (kf - cnt_gt) * kth) / kf


def kernel(logits, labels):
    out = pl.pallas_call(
        _ce_topk_kernel,
        grid=(_GB, _GH),
        in_specs=[
            pl.BlockSpec((1, _C, _HB, _W), lambda b, h: (b, 0, h, 0)),
            pl.BlockSpec((1, _HB, _W), lambda b, h: (b, h, 0)),
        ],
        out_specs=pl.BlockSpec(memory_space=pltpu.SMEM),
        out_shape=jax.ShapeDtypeStruct((1, 1), jnp.float32),
        scratch_shapes=[
            pltpu.VMEM((_SROWS, _SCOLS), jnp.float32),
            pltpu.VMEM((_SROWS, _SCOLS), jnp.int16),
        ],
    )(logits, labels)
    return out[0, 0]
